# baseline (speedup 1.0000x reference)
import jax
import jax.numpy as jnp
from jax import lax
from jax.experimental import pallas as pl
from jax.experimental.pallas import tpu as pltpu
from jax.experimental.pallas import tpu_sc as plsc

VOCAB = 100000
D_MODEL = 1024
BATCH = 4
SEQ = 4096
SCALE = 32.0

NC = 2
NS = 16
NW = NC * NS
LANES = 16

POS_PER_W = SEQ // NW
PC = 4
NCHUNK = POS_PER_W // PC
NG = 7
K = 5
NBUF = NG * BATCH
VPR = D_MODEL // LANES


def _sc_body(x_hbm, pe_hbm, table_hbm, out_hbm, *scr):
    idxa = scr[0]
    pe_v = scr[1:3]
    tb = scr[3:3 + NBUF]
    g = scr[3 + NBUF:3 + NBUF + NG]
    o = scr[3 + NBUF + NG:3 + NBUF + 2 * NG]
    q = scr[3 + NBUF + 2 * NG:3 + NBUF + 2 * NG + 2]

    wid = lax.axis_index("s") * NC + lax.axis_index("c")
    pos_base = wid * POS_PER_W

    idx_cps = [
        pltpu.async_copy(x_hbm.at[pl.ds(b * SEQ + pos_base, POS_PER_W)],
                         idxa.at[b], o[b])
        for b in range(BATCH)
    ]
    for cp in idx_cps:
        cp.wait()

    def start_gathers(c):
        grp = c % NG
        return [pltpu.async_copy(
            table_hbm.at[idxa.at[b, pl.ds(c * PC, PC)]],
            tb[grp * BATCH + b], g[grp]) for b in range(BATCH)]

    def start_pe(c):
        return pltpu.async_copy(pe_hbm.at[pl.ds(pos_base + c * PC, PC)],
                                pe_v[c % 2], q[c % 2])

    pending = {("q", 0): start_pe(0), ("q", 1): start_pe(1)}
    for c in range(K):
        pending[("g", c)] = start_gathers(c)

    for c in range(NCHUNK):
        grp = c % NG
        cn = c + K
        if cn < NCHUNK:
            co = cn - NG
            if co >= 0:
                for cp in pending.pop(("o", co)):
                    cp.wait()
            pending[("g", cn)] = start_gathers(cn)
        for cp in pending.pop(("g", c)):
            cp.wait()
        pending.pop(("q", c)).wait()
        bufs = tuple(tb[grp * BATCH + b] for b in range(BATCH))
        pe_b = pe_v[c % 2]

        @plsc.parallel_loop(0, PC * VPR, step=1, unroll=4)
        def fma_body(v, bufs=bufs, pe_b=pe_b):
            r = v >> 6
            sl = pl.ds(pl.multiple_of((v << 4) & (D_MODEL - 1), LANES), LANES)
            pv = pe_b[r, sl]
            for tbb in bufs:
                tbb[r, sl] = tbb[r, sl] * SCALE + pv

        if c + 2 < NCHUNK:
            pending[("q", c + 2)] = start_pe(c + 2)
        pending[("o", c)] = [pltpu.async_copy(
            bufs[b], out_hbm.at[pl.ds(b * SEQ + pos_base + c * PC, PC)],
            o[grp]) for b in range(BATCH)]

    for c in range(NCHUNK):
        if ("o", c) in pending:
            for cp in pending.pop(("o", c)):
                cp.wait()


@jax.jit
def _embed(x_flat, table, pe_flat):
    mesh = plsc.VectorSubcoreMesh(core_axis_name="c", subcore_axis_name="s")
    out = pl.kernel(
        _sc_body,
        out_type=jax.ShapeDtypeStruct((BATCH * SEQ, D_MODEL), jnp.float32),
        mesh=mesh,
        scratch_types=(
            [pltpu.VMEM((BATCH, POS_PER_W), jnp.int32)]
            + [pltpu.VMEM((PC, D_MODEL), jnp.float32)
               for _ in range(2 + NBUF)]
            + [pltpu.SemaphoreType.DMA for _ in range(2 * NG + 2)]
        ),
    )(x_flat, pe_flat, table)
    return out


def kernel(x, table, pe):
    x_flat = x.reshape(BATCH * SEQ).astype(jnp.int32)
    pe_flat = pe.reshape(-1, D_MODEL)[:SEQ]
    out = _embed(x_flat, table, pe_flat)
    return out.reshape(BATCH, SEQ, D_MODEL)


# speedup vs baseline: 1.0324x; 1.0324x over previous
import jax
import jax.numpy as jnp
from jax import lax
from jax.experimental import pallas as pl
from jax.experimental.pallas import tpu as pltpu
from jax.experimental.pallas import tpu_sc as plsc

VOCAB = 100000
D_MODEL = 1024
BATCH = 4
SEQ = 4096
SCALE = 32.0

NC = 2
NS = 16
NW = NC * NS
LANES = 16

POS_PER_W = SEQ // NW
PC = 8
NCHUNK = POS_PER_W // PC
NG = 3
NBUF = NG * BATCH
VPR = D_MODEL // LANES


def _sc_body(x_hbm, pe_hbm, table_hbm, out_hbm, *scr):
    idxa = scr[0]
    pe_v = scr[1:3]
    tb = scr[3:3 + NBUF]
    g = scr[3 + NBUF:3 + 2 * NBUF]
    o = scr[3 + 2 * NBUF:3 + 3 * NBUF]
    q = scr[3 + 3 * NBUF:3 + 3 * NBUF + 2]

    wid = lax.axis_index("s") * NC + lax.axis_index("c")
    pos_base = wid * POS_PER_W

    idx_cps = [
        pltpu.async_copy(x_hbm.at[pl.ds(b * SEQ + pos_base, POS_PER_W)],
                         idxa.at[b], o[b])
        for b in range(BATCH)
    ]
    for cp in idx_cps:
        cp.wait()

    def start_gather(c, b):
        slot = (c % NG) * BATCH + b
        idx_ref = idxa.at[b, pl.ds(c * PC, PC)]
        return pltpu.async_copy(table_hbm.at[idx_ref], tb[slot], g[slot])

    def start_pe(c):
        return pltpu.async_copy(pe_hbm.at[pl.ds(pos_base + c * PC, PC)],
                                pe_v[c % 2], q[c % 2])

    pending = {("q", 0): start_pe(0), ("q", 1): start_pe(1)}
    for c in range(NG):
        for b in range(BATCH):
            pending[("g", c, b)] = start_gather(c, b)

    for c in range(NCHUNK):
        grp = c % NG
        for b in range(BATCH):
            pending.pop(("g", c, b)).wait()
        pending.pop(("q", c)).wait()
        bufs = tuple(tb[grp * BATCH + b] for b in range(BATCH))
        pe_b = pe_v[c % 2]

        @plsc.parallel_loop(0, PC * VPR, step=1, unroll=4)
        def fma_body(v, bufs=bufs, pe_b=pe_b):
            r = v >> 6
            sl = pl.ds(pl.multiple_of((v << 4) & (D_MODEL - 1), LANES), LANES)
            pv = pe_b[r, sl]
            for tbb in bufs:
                tbb[r, sl] = tbb[r, sl] * SCALE + pv

        if c + 2 < NCHUNK:
            pending[("q", c + 2)] = start_pe(c + 2)
        for b in range(BATCH):
            row0 = b * SEQ + pos_base + c * PC
            pending[("o", c, b)] = pltpu.async_copy(
                bufs[b], out_hbm.at[pl.ds(row0, PC)], o[grp * BATCH + b])
        cn = c + NG
        if cn < NCHUNK:
            for b in range(BATCH):
                pending.pop(("o", c, b)).wait()
                pending[("g", cn, b)] = start_gather(cn, b)

    for c in range(NCHUNK - NG, NCHUNK):
        for b in range(BATCH):
            pending.pop(("o", c, b)).wait()


@jax.jit
def _embed(x_flat, table, pe_flat):
    mesh = plsc.VectorSubcoreMesh(core_axis_name="c", subcore_axis_name="s")
    out = pl.kernel(
        _sc_body,
        out_type=jax.ShapeDtypeStruct((BATCH * SEQ, D_MODEL), jnp.float32),
        mesh=mesh,
        scratch_types=(
            [pltpu.VMEM((BATCH, POS_PER_W), jnp.int32)]
            + [pltpu.VMEM((PC, D_MODEL), jnp.float32)
               for _ in range(2 + NBUF)]
            + [pltpu.SemaphoreType.DMA for _ in range(2 * NBUF + 2)]
        ),
    )(x_flat, pe_flat, table)
    return out


def kernel(x, table, pe):
    x_flat = x.reshape(BATCH * SEQ).astype(jnp.int32)
    pe_flat = pe.reshape(-1, D_MODEL)[:SEQ]
    out = _embed(x_flat, table, pe_flat)
    return out.reshape(BATCH, SEQ, D_MODEL)
